# baseline (speedup 1.0000x reference)
import functools

import jax
import jax.numpy as jnp
from jax import lax
from jax.experimental import pallas as pl
from jax.experimental.pallas import tpu as pltpu
from jax.experimental.pallas import tpu_sc as plsc

B = 16384
T = 12
D = 24
V = 28
NC = 2
NS = 16
NW = NC * NS
BPW = B // NW
RPW = BPW * T
CB = 128
NCHUNK = BPW // CB


def _dpat():
    lane = lax.iota(jnp.int32, 16)
    d0 = lane
    d1 = jnp.where(lane < 8, lane + 16, lane - 8)
    d2 = lane + 8
    return d0, d1, d2


def _body(
    x_hbm, w_hbm, p_hbm, out_hbm, x_v, w_v, p_v, c_v, idx_v, g_v, c_sh, sem0, sem1
):
    cid = lax.axis_index("c")
    sid = lax.axis_index("s")
    wid = sid * NC + cid
    bbase = wid * BPW

    pltpu.sync_copy(x_hbm.at[pl.ds(wid * RPW, RPW)], x_v)

    @pl.when(sid == 0)
    def _build():
        pltpu.sync_copy(w_hbm, w_v)
        pltpu.sync_copy(p_hbm, p_v)
        lane = lax.iota(jnp.int32, 16)
        dpat = _dpat()
        tadd = (lane * 0, (lane >= 8).astype(jnp.int32), lane * 0 + 1)

        def build(w, carry):
            dd, tt = carry
            for j in range(T * D // 16):
                m, r = divmod(j, 3)
                pv = p_v[pl.ds(16 * j, 16)]
                wv = plsc.load_gather(w_v, [w * D + dd[r]])
                plsc.store_scatter(c_v, [w * T + 2 * m + tt[r], dd[r]], wv + pv)
            return carry

        lax.fori_loop(0, V, build, (dpat, tadd))
        pltpu.sync_copy(c_v, c_sh)

    plsc.subcore_barrier()

    lane = lax.iota(jnp.int32, 16)
    tpat = (
        jnp.where(lane >= 12, lane - 12, lane),
        jnp.where(lane >= 8, lane - 8, lane + 4),
        jnp.where(lane >= 4, lane - 4, lane + 8),
    )
    bpat = (
        (lane >= 12).astype(jnp.int32),
        (lane >= 8).astype(jnp.int32) + 1,
        (lane >= 4).astype(jnp.int32) + 2,
    )

    def mkidx(i, carry):
        tt, bb = carry
        for r in range(3):
            xv = x_v[pl.ds((3 * i + r) * 16, 16)]
            plsc.store_scatter(idx_v, [tt[r], 4 * i + bb[r]], xv * T + tt[r])
        return carry

    lax.fori_loop(0, RPW // 48, mkidx, (tpat, bpat))

    ocopies = [None, None]
    for k in range(NCHUNK):
        buf = k % 2
        if ocopies[buf] is not None:
            for oc in ocopies[buf]:
                oc.wait()
        gcopies = []
        for t in range(T):
            gcopies.append(
                pltpu.async_copy(
                    c_sh.at[idx_v.at[t, pl.ds(k * CB, CB)]],
                    g_v.at[buf, t],
                    sem0,
                )
            )
        for gc in gcopies:
            gc.wait()
        ocopies[buf] = [
            pltpu.async_copy(
                g_v.at[buf, t],
                out_hbm.at[pl.ds(bbase + k * CB, CB), t],
                sem1,
            )
            for t in range(T)
        ]
    for bufcopies in ocopies:
        for oc in bufcopies:
            oc.wait()


_mesh = plsc.VectorSubcoreMesh(core_axis_name="c", subcore_axis_name="s")

_ebd = functools.partial(
    pl.kernel,
    mesh=_mesh,
    compiler_params=pltpu.CompilerParams(
        needs_layout_passes=False, use_tc_tiling_on_sc=False
    ),
    out_type=jax.ShapeDtypeStruct((B, T, D), jnp.float32),
    scratch_types=[
        pltpu.VMEM((RPW,), jnp.int32),
        pltpu.VMEM((V * D,), jnp.float32),
        pltpu.VMEM((T * D,), jnp.float32),
        pltpu.VMEM((V * T, D), jnp.float32),
        pltpu.VMEM((T, BPW), jnp.int32),
        pltpu.VMEM((2, T, CB, D), jnp.float32),
        pltpu.VMEM_SHARED((V * T, D), jnp.float32),
        pltpu.SemaphoreType.DMA,
        pltpu.SemaphoreType.DMA,
    ],
)(_body)


@jax.jit
def kernel(X, word_table, pos_table):
    return _ebd(X.reshape(-1), word_table.reshape(-1), pos_table.reshape(-1))


# speedup vs baseline: 1.0368x; 1.0368x over previous
import functools

import jax
import jax.numpy as jnp
from jax import lax
from jax.experimental import pallas as pl
from jax.experimental.pallas import tpu as pltpu
from jax.experimental.pallas import tpu_sc as plsc

B = 16384
T = 12
D = 24
V = 28
NC = 2
NS = 16
NW = NC * NS
RPW = (B * T) // NW
NV = RPW // 16
CHUNK = 2048
NCHUNK = RPW // CHUNK


def _tpat():
    lane = lax.iota(jnp.int32, 16)
    t0 = jnp.where(lane >= 12, lane - 12, lane)
    t1 = jnp.where(lane >= 8, lane - 8, lane + 4)
    t2 = jnp.where(lane >= 4, lane - 4, lane + 8)
    return t0, t1, t2


def _dpat():
    lane = lax.iota(jnp.int32, 16)
    d0 = lane
    d1 = jnp.where(lane < 8, lane + 16, lane - 8)
    d2 = lane + 8
    return d0, d1, d2


def _body(
    x_hbm, w_hbm, p_hbm, out_hbm, x_v, w_v, p_v, c_v, idx_v, g_v, c_sh,
    sem0, sem1, semx
):
    cid = lax.axis_index("c")
    sid = lax.axis_index("s")
    wid = sid * NC + cid
    xbase = wid * RPW

    xcopy = pltpu.async_copy(x_hbm.at[pl.ds(xbase, RPW)], x_v, semx)

    pltpu.sync_copy(w_hbm, w_v)
    pltpu.sync_copy(p_hbm, p_v)
    lane = lax.iota(jnp.int32, 16)
    dpat = _dpat()
    tadd = (lane * 0, (lane >= 8).astype(jnp.int32), lane * 0 + 1)

    for half in range(2):
        w = sid + NS * half

        @pl.when(w < V)
        def _build(w=w):
            for j in range(T * D // 16):
                m, r = divmod(j, 3)
                pv = p_v[pl.ds(16 * j, 16)]
                wv = plsc.load_gather(w_v, [w * D + dpat[r]])
                plsc.store_scatter(
                    c_v, [w * T + 2 * m + tadd[r], dpat[r]], wv + pv
                )
            pltpu.sync_copy(
                c_v.at[pl.ds(w * T, T)], c_sh.at[pl.ds(w * T, T)]
            )

    plsc.subcore_barrier()

    xcopy.wait()
    tpat = _tpat()

    def mkidx(i, carry):
        tt = carry
        for r in range(3):
            xv = x_v[pl.ds((3 * i + r) * 16, 16)]
            idx_v[pl.ds((3 * i + r) * 16, 16)] = xv * T + tt[r]
        return carry

    lax.fori_loop(0, NV // 3, mkidx, tpat)

    ocopies = [None, None]
    for k in range(NCHUNK):
        buf = k % 2
        if ocopies[buf] is not None:
            ocopies[buf].wait()
        pltpu.async_copy(
            c_sh.at[idx_v.at[pl.ds(k * CHUNK, CHUNK)]],
            g_v.at[buf],
            sem0,
        ).wait()
        ocopies[buf] = pltpu.async_copy(
            g_v.at[buf],
            out_hbm.at[pl.ds(xbase + k * CHUNK, CHUNK)],
            sem1,
        )
    ocopies[0].wait()
    ocopies[1].wait()


_mesh = plsc.VectorSubcoreMesh(core_axis_name="c", subcore_axis_name="s")

_ebd = functools.partial(
    pl.kernel,
    mesh=_mesh,
    compiler_params=pltpu.CompilerParams(
        needs_layout_passes=False, use_tc_tiling_on_sc=False
    ),
    out_type=jax.ShapeDtypeStruct((B * T, D), jnp.float32),
    scratch_types=[
        pltpu.VMEM((RPW,), jnp.int32),
        pltpu.VMEM((V * D,), jnp.float32),
        pltpu.VMEM((T * D,), jnp.float32),
        pltpu.VMEM((V * T, D), jnp.float32),
        pltpu.VMEM((RPW,), jnp.int32),
        pltpu.VMEM((2, CHUNK, D), jnp.float32),
        pltpu.VMEM_SHARED((V * T, D), jnp.float32),
        pltpu.SemaphoreType.DMA,
        pltpu.SemaphoreType.DMA,
        pltpu.SemaphoreType.DMA,
    ],
)(_body)


@jax.jit
def kernel(X, word_table, pos_table):
    out = _ebd(X.reshape(-1), word_table.reshape(-1), pos_table.reshape(-1))
    return out.reshape(B, T, D)


# speedup vs baseline: 1.0378x; 1.0009x over previous
import functools

import jax
import jax.numpy as jnp
from jax import lax
from jax.experimental import pallas as pl
from jax.experimental.pallas import tpu as pltpu
from jax.experimental.pallas import tpu_sc as plsc

B = 16384
T = 12
D = 24
V = 28
NC = 2
NS = 16
NW = NC * NS
RPW = (B * T) // NW
NV = RPW // 16
CHUNK = 2048
NCHUNK = RPW // CHUNK


def _tpat():
    lane = lax.iota(jnp.int32, 16)
    t0 = jnp.where(lane >= 12, lane - 12, lane)
    t1 = jnp.where(lane >= 8, lane - 8, lane + 4)
    t2 = jnp.where(lane >= 4, lane - 4, lane + 8)
    return t0, t1, t2


def _dpat():
    lane = lax.iota(jnp.int32, 16)
    d0 = lane
    d1 = jnp.where(lane < 8, lane + 16, lane - 8)
    d2 = lane + 8
    return d0, d1, d2


def _body(
    x_hbm, w_hbm, p_hbm, out_hbm, x_v, w_v, p_v, c_v, idx_v, g_v, c_sh,
    sem0, sem1, semx
):
    cid = lax.axis_index("c")
    sid = lax.axis_index("s")
    wid = sid * NC + cid
    xbase = wid * RPW

    xcopy = pltpu.async_copy(x_hbm.at[pl.ds(xbase, RPW)], x_v, semx)

    pltpu.sync_copy(w_hbm, w_v)
    pltpu.sync_copy(p_hbm, p_v)
    lane = lax.iota(jnp.int32, 16)
    dpat = _dpat()
    tadd = (lane * 0, (lane >= 8).astype(jnp.int32), lane * 0 + 1)

    for half in range(2):
        w = sid + NS * half

        @pl.when(w < V)
        def _build(w=w):
            for j in range(T * D // 16):
                m, r = divmod(j, 3)
                pv = p_v[pl.ds(16 * j, 16)]
                wv = plsc.load_gather(w_v, [w * D + dpat[r]])
                plsc.store_scatter(
                    c_v, [w * T + 2 * m + tadd[r], dpat[r]], wv + pv
                )
            pltpu.sync_copy(
                c_v.at[pl.ds(w * T, T)], c_sh.at[pl.ds(w * T, T)]
            )

    xcopy.wait()
    tpat = _tpat()

    def mkidx(i, carry):
        tt = carry
        for r in range(3):
            xv = x_v[pl.ds((3 * i + r) * 16, 16)]
            idx_v[pl.ds((3 * i + r) * 16, 16)] = xv * T + tt[r]
        return carry

    lax.fori_loop(0, NV // 3, mkidx, tpat)

    plsc.subcore_barrier()

    ocopies = [None, None]
    for k in range(NCHUNK):
        buf = k % 2
        if ocopies[buf] is not None:
            ocopies[buf].wait()
        pltpu.async_copy(
            c_sh.at[idx_v.at[pl.ds(k * CHUNK, CHUNK)]],
            g_v.at[buf],
            sem0,
        ).wait()
        ocopies[buf] = pltpu.async_copy(
            g_v.at[buf],
            out_hbm.at[pl.ds(xbase + k * CHUNK, CHUNK)],
            sem1,
        )
    ocopies[0].wait()
    ocopies[1].wait()


_mesh = plsc.VectorSubcoreMesh(core_axis_name="c", subcore_axis_name="s")

_ebd = functools.partial(
    pl.kernel,
    mesh=_mesh,
    compiler_params=pltpu.CompilerParams(
        needs_layout_passes=False, use_tc_tiling_on_sc=False
    ),
    out_type=jax.ShapeDtypeStruct((B * T, D), jnp.float32),
    scratch_types=[
        pltpu.VMEM((RPW,), jnp.int32),
        pltpu.VMEM((V * D,), jnp.float32),
        pltpu.VMEM((T * D,), jnp.float32),
        pltpu.VMEM((V * T, D), jnp.float32),
        pltpu.VMEM((RPW,), jnp.int32),
        pltpu.VMEM((2, CHUNK, D), jnp.float32),
        pltpu.VMEM_SHARED((V * T, D), jnp.float32),
        pltpu.SemaphoreType.DMA,
        pltpu.SemaphoreType.DMA,
        pltpu.SemaphoreType.DMA,
    ],
)(_body)


@jax.jit
def kernel(X, word_table, pos_table):
    out = _ebd(X.reshape(-1), word_table.reshape(-1), pos_table.reshape(-1))
    return out.reshape(B, T, D)
